# baseline (speedup 1.0000x reference)
import jax
import jax.numpy as jnp
from jax import lax
from jax.experimental import pallas as pl
from jax.experimental.pallas import tpu as pltpu
from jax.experimental.pallas import tpu_sc as plsc

N = 10000
E = 320000
D = 128
H = D // 2
O = 16

NC = 2
NS = 16
NW = NC * NS
C = 128
CH = 80
EW = C * CH
EPAD = NW * EW
NPAD = 10240
RPT = NPAD // NS
ZB = 128
BN = 1000


def _sc_segsum_body(ya, yb, srcx, dstx, part,
                    src_v, dst_v, rows_v, zb_v, acc_sh, *sems):
    cid = lax.axis_index("c")
    sid = lax.axis_index("s")
    wid = cid * NS + sid

    pltpu.sync_copy(srcx.at[wid], src_v)
    pltpu.sync_copy(dstx.at[wid], dst_v)

    z16 = jnp.zeros((16,), jnp.float32)

    def zrow(i, _):
        for k in range(H // 16):
            zb_v[i, pl.ds(k * 16, 16)] = z16
        return 0
    lax.fori_loop(0, ZB, zrow, 0)

    for h, y in ((0, ya), (1, yb)):
        for k in range(RPT // ZB):
            pltpu.sync_copy(zb_v, acc_sh.at[pl.ds(sid * RPT + k * ZB, ZB)])
        plsc.subcore_barrier()

        def step(j, _):
            pltpu.sync_copy(y.at[src_v.at[j]], rows_v)
            pltpu.sync_copy(rows_v, acc_sh.at[dst_v.at[j]], add=True)
            return 0
        lax.fori_loop(0, CH, step, 0)
        plsc.subcore_barrier()

        sl = pl.ds(sid * RPT, RPT)
        pltpu.sync_copy(acc_sh.at[sl], part.at[cid, h, sl])
        plsc.subcore_barrier()


_sc_segsum = pl.kernel(
    _sc_segsum_body,
    out_type=[jax.ShapeDtypeStruct((NC, 2, NPAD, H), jnp.float32)],
    mesh=plsc.VectorSubcoreMesh(core_axis_name="c", subcore_axis_name="s"),
    scratch_types=[
        pltpu.VMEM((CH, C), jnp.int32),
        pltpu.VMEM((CH, C), jnp.int32),
        pltpu.VMEM((C, H), jnp.float32),
        pltpu.VMEM((ZB, H), jnp.float32),
        pltpu.VMEM_SHARED((NPAD, H), jnp.float32),
    ],
    compiler_params=pltpu.CompilerParams(use_tc_tiling_on_sc=False),
)


def _sc_deg_body(dstx, deg, dst_v, zd_v, ones_v, deg_sh):
    cid = lax.axis_index("c")
    sid = lax.axis_index("s")
    wid = cid * NS + sid

    pltpu.sync_copy(dstx.at[wid], dst_v)

    z16 = jnp.zeros((16,), jnp.float32)
    o16 = jnp.ones((16,), jnp.float32)

    def zdrow(i, _):
        zd_v[i, pl.ds(0, 16)] = z16
        return 0
    lax.fori_loop(0, RPT, zdrow, 0)

    def orow(i, _):
        ones_v[i, pl.ds(0, 16)] = o16
        return 0
    lax.fori_loop(0, C, orow, 0)

    pltpu.sync_copy(zd_v, deg_sh.at[pl.ds(sid * RPT, RPT)])
    plsc.subcore_barrier()

    def step(j, _):
        pltpu.sync_copy(ones_v, deg_sh.at[dst_v.at[j]], add=True)
        return 0
    lax.fori_loop(0, CH, step, 0)
    plsc.subcore_barrier()

    sl = pl.ds(sid * RPT, RPT)
    pltpu.sync_copy(deg_sh.at[sl], deg.at[cid, sl])


_sc_deg = pl.kernel(
    _sc_deg_body,
    out_type=[jax.ShapeDtypeStruct((NC, NPAD, 16), jnp.float32)],
    mesh=plsc.VectorSubcoreMesh(core_axis_name="c", subcore_axis_name="s"),
    scratch_types=[
        pltpu.VMEM((CH, C), jnp.int32),
        pltpu.VMEM((RPT, 16), jnp.float32),
        pltpu.VMEM((C, 16), jnp.float32),
        pltpu.VMEM_SHARED((NPAD, 16), jnp.float32),
    ],
    compiler_params=pltpu.CompilerParams(use_tc_tiling_on_sc=False),
)


def _mm_body(x_ref, w_ref, ya, yb, r):
    yw = jnp.dot(x_ref[...], w_ref[...], preferred_element_type=jnp.float32)
    ya[...] = yw[:, :H]
    yb[...] = yw[:, H:D]
    r[...] = yw[:, D:]


def _matmul(x, w):
    return pl.pallas_call(
        _mm_body,
        grid=(N // BN,),
        in_specs=[pl.BlockSpec((BN, D), lambda i: (i, 0)),
                  pl.BlockSpec((D, 2 * D), lambda i: (0, 0))],
        out_specs=[pl.BlockSpec((BN, H), lambda i: (i, 0)),
                   pl.BlockSpec((BN, H), lambda i: (i, 0)),
                   pl.BlockSpec((BN, D), lambda i: (i, 0))],
        out_shape=[jax.ShapeDtypeStruct((N, H), jnp.float32),
                   jax.ShapeDtypeStruct((N, H), jnp.float32),
                   jax.ShapeDtypeStruct((N, D), jnp.float32)],
    )(x, w)


def _comb_mm_body(p0a, p0b, p1a, p1b, d0, d1, r1, b, w, ya, yb, r2):
    deg = jnp.maximum(d0[...] + d1[...], 1.0)[:, 0:1]
    ps = jnp.concatenate([p0a[...] + p1a[...], p0b[...] + p1b[...]], axis=1)
    hh = jnp.maximum(ps / deg + b[...] + r1[...], 0.0)
    yw = jnp.dot(hh, w[...], preferred_element_type=jnp.float32)
    ya[...] = yw[:, :H]
    yb[...] = yw[:, H:D]
    r2[...] = yw[:, D:]


def _combine_mm(p0a, p0b, p1a, p1b, d0, d1, r1, b, w):
    return pl.pallas_call(
        _comb_mm_body,
        grid=(N // BN,),
        in_specs=[pl.BlockSpec((BN, H), lambda i: (i, 0)),
                  pl.BlockSpec((BN, H), lambda i: (i, 0)),
                  pl.BlockSpec((BN, H), lambda i: (i, 0)),
                  pl.BlockSpec((BN, H), lambda i: (i, 0)),
                  pl.BlockSpec((BN, 16), lambda i: (i, 0)),
                  pl.BlockSpec((BN, 16), lambda i: (i, 0)),
                  pl.BlockSpec((BN, D), lambda i: (i, 0)),
                  pl.BlockSpec((1, D), lambda i: (0, 0)),
                  pl.BlockSpec((D, 2 * D), lambda i: (0, 0))],
        out_specs=[pl.BlockSpec((BN, H), lambda i: (i, 0)),
                   pl.BlockSpec((BN, H), lambda i: (i, 0)),
                   pl.BlockSpec((BN, D), lambda i: (i, 0))],
        out_shape=[jax.ShapeDtypeStruct((N, H), jnp.float32),
                   jax.ShapeDtypeStruct((N, H), jnp.float32),
                   jax.ShapeDtypeStruct((N, D), jnp.float32)],
    )(p0a, p0b, p1a, p1b, d0, d1, r1, b, w)


def _comb_pool_body(p0a, p0b, p1a, p1b, d0, d1, r2, b, wo, bo, out, acc):
    i = pl.program_id(0)
    deg = jnp.maximum(d0[...] + d1[...], 1.0)[:, 0:1]
    ps = jnp.concatenate([p0a[...] + p1a[...], p0b[...] + p1b[...]], axis=1)
    hh = jnp.maximum(ps / deg + b[...] + r2[...], 0.0)
    s = jnp.sum(hh, axis=0, keepdims=True)

    @pl.when(i == 0)
    def _():
        acc[...] = jnp.zeros_like(acc)

    acc[0:1, :] += s
    out[...] = (jnp.dot(acc[0:1, :] * (1.0 / N), wo[...],
                        preferred_element_type=jnp.float32) + bo[...])


def _combine_pool(p0a, p0b, p1a, p1b, d0, d1, r2, b, wo, bo):
    return pl.pallas_call(
        _comb_pool_body,
        grid=(N // BN,),
        in_specs=[pl.BlockSpec((BN, H), lambda i: (i, 0)),
                  pl.BlockSpec((BN, H), lambda i: (i, 0)),
                  pl.BlockSpec((BN, H), lambda i: (i, 0)),
                  pl.BlockSpec((BN, H), lambda i: (i, 0)),
                  pl.BlockSpec((BN, 16), lambda i: (i, 0)),
                  pl.BlockSpec((BN, 16), lambda i: (i, 0)),
                  pl.BlockSpec((BN, D), lambda i: (i, 0)),
                  pl.BlockSpec((1, D), lambda i: (0, 0)),
                  pl.BlockSpec((D, O), lambda i: (0, 0)),
                  pl.BlockSpec((1, O), lambda i: (0, 0))],
        out_specs=pl.BlockSpec((1, O), lambda i: (0, 0)),
        out_shape=jax.ShapeDtypeStruct((1, O), jnp.float32),
        scratch_shapes=[pltpu.VMEM((8, D), jnp.float32)],
    )(p0a, p0b, p1a, p1b, d0, d1, r2, b, wo, bo)


def kernel(x, edge_index, W1l, b1, W1r, W2l, b2, W2r, Wo, bo):
    src = edge_index[0].astype(jnp.int32)
    dst = edge_index[1].astype(jnp.int32)
    pad = EPAD - E
    srcx = jnp.concatenate([src, jnp.zeros((pad,), jnp.int32)]).reshape(NW, CH, C)
    dstx = jnp.concatenate([dst, jnp.full((pad,), N, jnp.int32)]).reshape(NW, CH, C)

    y1a, y1b, r1 = _matmul(x, jnp.concatenate([W1l, W1r], axis=1))
    (degs,) = _sc_deg(dstx)
    (parts1,) = _sc_segsum(y1a, y1b, srcx, dstx)
    y2a, y2b, r2 = _combine_mm(parts1[0, 0], parts1[0, 1],
                               parts1[1, 0], parts1[1, 1],
                               degs[0], degs[1],
                               r1, b1.reshape(1, D),
                               jnp.concatenate([W2l, W2r], axis=1))
    (parts2,) = _sc_segsum(y2a, y2b, srcx, dstx)
    return _combine_pool(parts2[0, 0], parts2[0, 1],
                         parts2[1, 0], parts2[1, 1],
                         degs[0], degs[1], r2,
                         b2.reshape(1, D), Wo, bo.reshape(1, O))


# speedup vs baseline: 1.3992x; 1.3992x over previous
import jax
import jax.numpy as jnp
from jax import lax
from jax.experimental import pallas as pl
from jax.experimental.pallas import tpu as pltpu
from jax.experimental.pallas import tpu_sc as plsc

N = 10000
E = 320000
D = 128
H = D // 2
O = 16

NC = 2
NS = 16
NW = NC * NS
C = 128
CH = 79
EW = C * CH
EPAD = NW * EW
NPAD = 10240
RPT = NPAD // NS
ZB = 128
BN = 1000


def _sc_segsum_body(ya, yb, srcx, dstx, part,
                    src_v, dst_v, rows_v, zb_v, acc_sh, *sems):
    cid = lax.axis_index("c")
    sid = lax.axis_index("s")
    wid = cid * NS + sid

    pltpu.sync_copy(srcx.at[wid], src_v)
    pltpu.sync_copy(dstx.at[wid], dst_v)

    z16 = jnp.zeros((16,), jnp.float32)

    def zrow(i, _):
        for k in range(H // 16):
            zb_v[i, pl.ds(k * 16, 16)] = z16
        return 0
    lax.fori_loop(0, ZB, zrow, 0)

    for h, y in ((0, ya), (1, yb)):
        for k in range(RPT // ZB):
            pltpu.sync_copy(zb_v, acc_sh.at[pl.ds(sid * RPT + k * ZB, ZB)])
        plsc.subcore_barrier()

        def step(j, _):
            pltpu.sync_copy(y.at[src_v.at[j]], rows_v)
            pltpu.sync_copy(rows_v, acc_sh.at[dst_v.at[j]], add=True)
            return 0
        lax.fori_loop(0, CH, step, 0)
        plsc.subcore_barrier()

        sl = pl.ds(sid * RPT, RPT)
        pltpu.sync_copy(acc_sh.at[sl], part.at[cid, h, sl])
        plsc.subcore_barrier()


_sc_segsum = pl.kernel(
    _sc_segsum_body,
    out_type=[jax.ShapeDtypeStruct((NC, 2, NPAD, H), jnp.float32)],
    mesh=plsc.VectorSubcoreMesh(core_axis_name="c", subcore_axis_name="s"),
    scratch_types=[
        pltpu.VMEM((CH, C), jnp.int32),
        pltpu.VMEM((CH, C), jnp.int32),
        pltpu.VMEM((C, H), jnp.float32),
        pltpu.VMEM((ZB, H), jnp.float32),
        pltpu.VMEM_SHARED((NPAD, H), jnp.float32),
    ],
    compiler_params=pltpu.CompilerParams(use_tc_tiling_on_sc=False),
)


def _sc_deg_body(dstx, deg, dst_v, zd_v, ones_v, deg_sh):
    cid = lax.axis_index("c")
    sid = lax.axis_index("s")
    wid = cid * NS + sid

    pltpu.sync_copy(dstx.at[wid], dst_v)

    z16 = jnp.zeros((16,), jnp.float32)
    o16 = jnp.ones((16,), jnp.float32)

    def zdrow(i, _):
        zd_v[i, pl.ds(0, 16)] = z16
        return 0
    lax.fori_loop(0, RPT, zdrow, 0)

    def orow(i, _):
        ones_v[i, pl.ds(0, 16)] = o16
        return 0
    lax.fori_loop(0, C, orow, 0)

    pltpu.sync_copy(zd_v, deg_sh.at[pl.ds(sid * RPT, RPT)])
    plsc.subcore_barrier()

    def step(j, _):
        pltpu.sync_copy(ones_v, deg_sh.at[dst_v.at[j]], add=True)
        return 0
    lax.fori_loop(0, CH, step, 0)
    plsc.subcore_barrier()

    sl = pl.ds(sid * RPT, RPT)
    pltpu.sync_copy(deg_sh.at[sl], deg.at[cid, sl])


_sc_deg = pl.kernel(
    _sc_deg_body,
    out_type=[jax.ShapeDtypeStruct((NC, NPAD, 16), jnp.float32)],
    mesh=plsc.VectorSubcoreMesh(core_axis_name="c", subcore_axis_name="s"),
    scratch_types=[
        pltpu.VMEM((CH, C), jnp.int32),
        pltpu.VMEM((RPT, 16), jnp.float32),
        pltpu.VMEM((C, 16), jnp.float32),
        pltpu.VMEM_SHARED((NPAD, 16), jnp.float32),
    ],
    compiler_params=pltpu.CompilerParams(use_tc_tiling_on_sc=False),
)


def _mm_body(x_ref, w_ref, ya, yb, r):
    yw = jnp.dot(x_ref[...], w_ref[...], preferred_element_type=jnp.float32)
    ya[...] = yw[:, :H]
    yb[...] = yw[:, H:D]
    r[...] = yw[:, D:]


def _matmul(x, w):
    return pl.pallas_call(
        _mm_body,
        grid=(N // BN,),
        in_specs=[pl.BlockSpec((BN, D), lambda i: (i, 0)),
                  pl.BlockSpec((D, 2 * D), lambda i: (0, 0))],
        out_specs=[pl.BlockSpec((BN, H), lambda i: (i, 0)),
                   pl.BlockSpec((BN, H), lambda i: (i, 0)),
                   pl.BlockSpec((BN, D), lambda i: (i, 0))],
        out_shape=[jax.ShapeDtypeStruct((N, H), jnp.float32),
                   jax.ShapeDtypeStruct((N, H), jnp.float32),
                   jax.ShapeDtypeStruct((N, D), jnp.float32)],
    )(x, w)


def _comb_mm_body(p0a, p0b, p1a, p1b, d0, d1, r1, b, w, ya, yb, r2):
    deg = jnp.maximum(d0[...] + d1[...], 1.0)[:, 0:1]
    ps = jnp.concatenate([p0a[...] + p1a[...], p0b[...] + p1b[...]], axis=1)
    hh = jnp.maximum(ps / deg + b[...] + r1[...], 0.0)
    yw = jnp.dot(hh, w[...], preferred_element_type=jnp.float32)
    ya[...] = yw[:, :H]
    yb[...] = yw[:, H:D]
    r2[...] = yw[:, D:]


def _combine_mm(p0a, p0b, p1a, p1b, d0, d1, r1, b, w):
    return pl.pallas_call(
        _comb_mm_body,
        grid=(N // BN,),
        in_specs=[pl.BlockSpec((BN, H), lambda i: (i, 0)),
                  pl.BlockSpec((BN, H), lambda i: (i, 0)),
                  pl.BlockSpec((BN, H), lambda i: (i, 0)),
                  pl.BlockSpec((BN, H), lambda i: (i, 0)),
                  pl.BlockSpec((BN, 16), lambda i: (i, 0)),
                  pl.BlockSpec((BN, 16), lambda i: (i, 0)),
                  pl.BlockSpec((BN, D), lambda i: (i, 0)),
                  pl.BlockSpec((1, D), lambda i: (0, 0)),
                  pl.BlockSpec((D, 2 * D), lambda i: (0, 0))],
        out_specs=[pl.BlockSpec((BN, H), lambda i: (i, 0)),
                   pl.BlockSpec((BN, H), lambda i: (i, 0)),
                   pl.BlockSpec((BN, D), lambda i: (i, 0))],
        out_shape=[jax.ShapeDtypeStruct((N, H), jnp.float32),
                   jax.ShapeDtypeStruct((N, H), jnp.float32),
                   jax.ShapeDtypeStruct((N, D), jnp.float32)],
    )(p0a, p0b, p1a, p1b, d0, d1, r1, b, w)


def _comb_pool_body(p0a, p0b, p1a, p1b, d0, d1, r2, b, wo, bo, out, acc):
    i = pl.program_id(0)
    deg = jnp.maximum(d0[...] + d1[...], 1.0)[:, 0:1]
    ps = jnp.concatenate([p0a[...] + p1a[...], p0b[...] + p1b[...]], axis=1)
    hh = jnp.maximum(ps / deg + b[...] + r2[...], 0.0)
    s = jnp.sum(hh, axis=0, keepdims=True)

    @pl.when(i == 0)
    def _():
        acc[...] = jnp.zeros_like(acc)

    acc[0:1, :] += s
    out[...] = (jnp.dot(acc[0:1, :] * (1.0 / N), wo[...],
                        preferred_element_type=jnp.float32) + bo[...])


def _combine_pool(p0a, p0b, p1a, p1b, d0, d1, r2, b, wo, bo):
    return pl.pallas_call(
        _comb_pool_body,
        grid=(N // BN,),
        in_specs=[pl.BlockSpec((BN, H), lambda i: (i, 0)),
                  pl.BlockSpec((BN, H), lambda i: (i, 0)),
                  pl.BlockSpec((BN, H), lambda i: (i, 0)),
                  pl.BlockSpec((BN, H), lambda i: (i, 0)),
                  pl.BlockSpec((BN, 16), lambda i: (i, 0)),
                  pl.BlockSpec((BN, 16), lambda i: (i, 0)),
                  pl.BlockSpec((BN, D), lambda i: (i, 0)),
                  pl.BlockSpec((1, D), lambda i: (0, 0)),
                  pl.BlockSpec((D, O), lambda i: (0, 0)),
                  pl.BlockSpec((1, O), lambda i: (0, 0))],
        out_specs=pl.BlockSpec((1, O), lambda i: (0, 0)),
        out_shape=jax.ShapeDtypeStruct((1, O), jnp.float32),
        scratch_shapes=[pltpu.VMEM((8, D), jnp.float32)],
    )(p0a, p0b, p1a, p1b, d0, d1, r2, b, wo, bo)


def kernel(x, edge_index, W1l, b1, W1r, W2l, b2, W2r, Wo, bo):
    src = edge_index[0].astype(jnp.int32)
    dst = edge_index[1].astype(jnp.int32)
    pad = EPAD - E
    srcx = jnp.concatenate([src, jnp.zeros((pad,), jnp.int32)]).reshape(NW, CH, C)
    dstx = jnp.concatenate([dst, jnp.full((pad,), N, jnp.int32)]).reshape(NW, CH, C)

    y1a, y1b, r1 = _matmul(x, jnp.concatenate([W1l, W1r], axis=1))
    (degs,) = _sc_deg(dstx)
    (parts1,) = _sc_segsum(y1a, y1b, srcx, dstx)
    y2a, y2b, r2 = _combine_mm(parts1[0, 0], parts1[0, 1],
                               parts1[1, 0], parts1[1, 1],
                               degs[0], degs[1],
                               r1, b1.reshape(1, D),
                               jnp.concatenate([W2l, W2r], axis=1))
    (parts2,) = _sc_segsum(y2a, y2b, srcx, dstx)
    return _combine_pool(parts2[0, 0], parts2[0, 1],
                         parts2[1, 0], parts2[1, 1],
                         degs[0], degs[1], r2,
                         b2.reshape(1, D), Wo, bo.reshape(1, O))


# speedup vs baseline: 2.0464x; 1.4626x over previous
import jax
import jax.numpy as jnp
from jax import lax
from jax.experimental import pallas as pl
from jax.experimental.pallas import tpu as pltpu
from jax.experimental.pallas import tpu_sc as plsc

N = 10000
E = 320000
D = 128
H = D // 2
O = 16

NC = 2
NS = 16
NW = NC * NS
C = 128
CH = 80
EW = C * CH
EPAD = NW * EW
NPAD = 10240
RPT = NPAD // NS
ZB = 128
BN = 1000


def _sc_segsum_body(ya, yb, srcx, dstx, part,
                    src_v, dst_v, rows_v, zb_v, acc_sh, *sems):
    cid = lax.axis_index("c")
    sid = lax.axis_index("s")
    wid = cid * NS + sid

    pltpu.sync_copy(srcx.at[wid], src_v)
    pltpu.sync_copy(dstx.at[wid], dst_v)

    z16 = jnp.zeros((16,), jnp.float32)

    def zrow(i, _):
        for k in range(H // 16):
            zb_v[i, pl.ds(k * 16, 16)] = z16
        return 0
    lax.fori_loop(0, ZB, zrow, 0)

    for h, y in ((0, ya), (1, yb)):
        for k in range(RPT // ZB):
            pltpu.sync_copy(zb_v, acc_sh.at[pl.ds(sid * RPT + k * ZB, ZB)])
        plsc.subcore_barrier()

        def step(j, _):
            pltpu.sync_copy(y.at[src_v.at[j]], rows_v)
            pltpu.sync_copy(rows_v, acc_sh.at[dst_v.at[j]], add=True)
            return 0
        lax.fori_loop(0, CH, step, 0)
        plsc.subcore_barrier()

        sl = pl.ds(sid * RPT, RPT)
        pltpu.sync_copy(acc_sh.at[sl], part.at[cid, h, sl])
        plsc.subcore_barrier()


_sc_segsum = pl.kernel(
    _sc_segsum_body,
    out_type=[jax.ShapeDtypeStruct((NC, 2, NPAD, H), jnp.float32)],
    mesh=plsc.VectorSubcoreMesh(core_axis_name="c", subcore_axis_name="s"),
    scratch_types=[
        pltpu.VMEM((CH, C), jnp.int32),
        pltpu.VMEM((CH, C), jnp.int32),
        pltpu.VMEM((C, H), jnp.float32),
        pltpu.VMEM((ZB, H), jnp.float32),
        pltpu.VMEM_SHARED((NPAD, H), jnp.float32),
    ],
    compiler_params=pltpu.CompilerParams(use_tc_tiling_on_sc=False),
)


def _sc_deg_body(dstx, deg, dst_v, zd_v, ones_v, deg_sh):
    cid = lax.axis_index("c")
    sid = lax.axis_index("s")
    wid = cid * NS + sid

    pltpu.sync_copy(dstx.at[wid], dst_v)

    z16 = jnp.zeros((16,), jnp.float32)
    o16 = jnp.ones((16,), jnp.float32)

    def zdrow(i, _):
        zd_v[i, pl.ds(0, 16)] = z16
        return 0
    lax.fori_loop(0, RPT, zdrow, 0)

    def orow(i, _):
        ones_v[i, pl.ds(0, 16)] = o16
        return 0
    lax.fori_loop(0, C, orow, 0)

    pltpu.sync_copy(zd_v, deg_sh.at[pl.ds(sid * RPT, RPT)])
    plsc.subcore_barrier()

    def step(j, _):
        pltpu.sync_copy(ones_v, deg_sh.at[dst_v.at[j]], add=True)
        return 0
    lax.fori_loop(0, CH, step, 0)
    plsc.subcore_barrier()

    sl = pl.ds(sid * RPT, RPT)
    pltpu.sync_copy(deg_sh.at[sl], deg.at[cid, sl])


_sc_deg = pl.kernel(
    _sc_deg_body,
    out_type=[jax.ShapeDtypeStruct((NC, NPAD, 16), jnp.float32)],
    mesh=plsc.VectorSubcoreMesh(core_axis_name="c", subcore_axis_name="s"),
    scratch_types=[
        pltpu.VMEM((CH, C), jnp.int32),
        pltpu.VMEM((RPT, 16), jnp.float32),
        pltpu.VMEM((C, 16), jnp.float32),
        pltpu.VMEM_SHARED((NPAD, 16), jnp.float32),
    ],
    compiler_params=pltpu.CompilerParams(use_tc_tiling_on_sc=False),
)


def _mm_body(x_ref, w_ref, ya, yb, r):
    yw = jnp.dot(x_ref[...], w_ref[...], preferred_element_type=jnp.float32)
    ya[...] = yw[:, :H]
    yb[...] = yw[:, H:D]
    r[...] = yw[:, D:]


def _matmul(x, w):
    return pl.pallas_call(
        _mm_body,
        grid=(N // BN,),
        in_specs=[pl.BlockSpec((BN, D), lambda i: (i, 0)),
                  pl.BlockSpec((D, 2 * D), lambda i: (0, 0))],
        out_specs=[pl.BlockSpec((BN, H), lambda i: (i, 0)),
                   pl.BlockSpec((BN, H), lambda i: (i, 0)),
                   pl.BlockSpec((BN, D), lambda i: (i, 0))],
        out_shape=[jax.ShapeDtypeStruct((N, H), jnp.float32),
                   jax.ShapeDtypeStruct((N, H), jnp.float32),
                   jax.ShapeDtypeStruct((N, D), jnp.float32)],
    )(x, w)


def _comb_mm_body(p0a, p0b, p1a, p1b, d0, d1, r1, b, w, ya, yb, r2):
    deg = jnp.maximum(d0[...] + d1[...], 1.0)[:, 0:1]
    ps = jnp.concatenate([p0a[...] + p1a[...], p0b[...] + p1b[...]], axis=1)
    hh = jnp.maximum(ps / deg + b[...] + r1[...], 0.0)
    yw = jnp.dot(hh, w[...], preferred_element_type=jnp.float32)
    ya[...] = yw[:, :H]
    yb[...] = yw[:, H:D]
    r2[...] = yw[:, D:]


def _combine_mm(p0a, p0b, p1a, p1b, d0, d1, r1, b, w):
    return pl.pallas_call(
        _comb_mm_body,
        grid=(N // BN,),
        in_specs=[pl.BlockSpec((BN, H), lambda i: (i, 0)),
                  pl.BlockSpec((BN, H), lambda i: (i, 0)),
                  pl.BlockSpec((BN, H), lambda i: (i, 0)),
                  pl.BlockSpec((BN, H), lambda i: (i, 0)),
                  pl.BlockSpec((BN, 16), lambda i: (i, 0)),
                  pl.BlockSpec((BN, 16), lambda i: (i, 0)),
                  pl.BlockSpec((BN, D), lambda i: (i, 0)),
                  pl.BlockSpec((1, D), lambda i: (0, 0)),
                  pl.BlockSpec((D, 2 * D), lambda i: (0, 0))],
        out_specs=[pl.BlockSpec((BN, H), lambda i: (i, 0)),
                   pl.BlockSpec((BN, H), lambda i: (i, 0)),
                   pl.BlockSpec((BN, D), lambda i: (i, 0))],
        out_shape=[jax.ShapeDtypeStruct((N, H), jnp.float32),
                   jax.ShapeDtypeStruct((N, H), jnp.float32),
                   jax.ShapeDtypeStruct((N, D), jnp.float32)],
    )(p0a, p0b, p1a, p1b, d0, d1, r1, b, w)


def _comb_pool_body(p0a, p0b, p1a, p1b, d0, d1, r2, b, wo, bo, out, acc):
    i = pl.program_id(0)
    deg = jnp.maximum(d0[...] + d1[...], 1.0)[:, 0:1]
    ps = jnp.concatenate([p0a[...] + p1a[...], p0b[...] + p1b[...]], axis=1)
    hh = jnp.maximum(ps / deg + b[...] + r2[...], 0.0)
    s = jnp.sum(hh, axis=0, keepdims=True)

    @pl.when(i == 0)
    def _():
        acc[...] = jnp.zeros_like(acc)

    acc[0:1, :] += s
    out[...] = (jnp.dot(acc[0:1, :] * (1.0 / N), wo[...],
                        preferred_element_type=jnp.float32) + bo[...])


def _combine_pool(p0a, p0b, p1a, p1b, d0, d1, r2, b, wo, bo):
    return pl.pallas_call(
        _comb_pool_body,
        grid=(N // BN,),
        in_specs=[pl.BlockSpec((BN, H), lambda i: (i, 0)),
                  pl.BlockSpec((BN, H), lambda i: (i, 0)),
                  pl.BlockSpec((BN, H), lambda i: (i, 0)),
                  pl.BlockSpec((BN, H), lambda i: (i, 0)),
                  pl.BlockSpec((BN, 16), lambda i: (i, 0)),
                  pl.BlockSpec((BN, 16), lambda i: (i, 0)),
                  pl.BlockSpec((BN, D), lambda i: (i, 0)),
                  pl.BlockSpec((1, D), lambda i: (0, 0)),
                  pl.BlockSpec((D, O), lambda i: (0, 0)),
                  pl.BlockSpec((1, O), lambda i: (0, 0))],
        out_specs=pl.BlockSpec((1, O), lambda i: (0, 0)),
        out_shape=jax.ShapeDtypeStruct((1, O), jnp.float32),
        scratch_shapes=[pltpu.VMEM((8, D), jnp.float32)],
    )(p0a, p0b, p1a, p1b, d0, d1, r2, b, wo, bo)


def kernel(x, edge_index, W1l, b1, W1r, W2l, b2, W2r, Wo, bo):
    src = edge_index[0].astype(jnp.int32)
    dst = edge_index[1].astype(jnp.int32)
    pad = EPAD - E
    srcx = jnp.concatenate([src, jnp.arange(pad, dtype=jnp.int32)]).reshape(NW, CH, C)
    dstx = jnp.concatenate([dst, jnp.full((pad,), N, jnp.int32)]).reshape(NW, CH, C)

    y1a, y1b, r1 = _matmul(x, jnp.concatenate([W1l, W1r], axis=1))
    (degs,) = _sc_deg(dstx)
    (parts1,) = _sc_segsum(y1a, y1b, srcx, dstx)
    y2a, y2b, r2 = _combine_mm(parts1[0, 0], parts1[0, 1],
                               parts1[1, 0], parts1[1, 1],
                               degs[0], degs[1],
                               r1, b1.reshape(1, D),
                               jnp.concatenate([W2l, W2r], axis=1))
    (parts2,) = _sc_segsum(y2a, y2b, srcx, dstx)
    return _combine_pool(parts2[0, 0], parts2[0, 1],
                         parts2[1, 0], parts2[1, 1],
                         degs[0], degs[1], r2,
                         b2.reshape(1, D), Wo, bo.reshape(1, O))


# speedup vs baseline: 2.4591x; 1.2016x over previous
import jax
import jax.numpy as jnp
from jax import lax
from jax.experimental import pallas as pl
from jax.experimental.pallas import tpu as pltpu
from jax.experimental.pallas import tpu_sc as plsc

N = 10000
E = 320000
D = 128
H = D // 2
O = 16

NC = 2
NS = 16
NW = NC * NS
C = 256
CH = 40
EW = C * CH
EPAD = NW * EW
NPAD = 10240
RPT = NPAD // NS
ZB = 128
BN = 1000


def _sc_segsum_body(ya, yb, srcx, dstx, part,
                    src_v, dst_v, rows_v, zb_v, acc_sh, *sems):
    cid = lax.axis_index("c")
    sid = lax.axis_index("s")
    wid = cid * NS + sid

    pltpu.sync_copy(srcx.at[wid], src_v)
    pltpu.sync_copy(dstx.at[wid], dst_v)

    z16 = jnp.zeros((16,), jnp.float32)

    def zrow(i, _):
        for k in range(H // 16):
            zb_v[i, pl.ds(k * 16, 16)] = z16
        return 0
    lax.fori_loop(0, ZB, zrow, 0)

    for h, y in ((0, ya), (1, yb)):
        for k in range(RPT // ZB):
            pltpu.sync_copy(zb_v, acc_sh.at[pl.ds(sid * RPT + k * ZB, ZB)])
        plsc.subcore_barrier()

        def step(j, _):
            pltpu.sync_copy(y.at[src_v.at[j]], rows_v)
            pltpu.sync_copy(rows_v, acc_sh.at[dst_v.at[j]], add=True)
            return 0
        lax.fori_loop(0, CH, step, 0)
        plsc.subcore_barrier()

        sl = pl.ds(sid * RPT, RPT)
        pltpu.sync_copy(acc_sh.at[sl], part.at[cid, h, sl])
        plsc.subcore_barrier()


_sc_segsum = pl.kernel(
    _sc_segsum_body,
    out_type=[jax.ShapeDtypeStruct((NC, 2, NPAD, H), jnp.float32)],
    mesh=plsc.VectorSubcoreMesh(core_axis_name="c", subcore_axis_name="s"),
    scratch_types=[
        pltpu.VMEM((CH, C), jnp.int32),
        pltpu.VMEM((CH, C), jnp.int32),
        pltpu.VMEM((C, H), jnp.float32),
        pltpu.VMEM((ZB, H), jnp.float32),
        pltpu.VMEM_SHARED((NPAD, H), jnp.float32),
    ],
    compiler_params=pltpu.CompilerParams(use_tc_tiling_on_sc=False),
)


def _sc_deg_body(dstx, deg, dst_v, zd_v, ones_v, deg_sh):
    cid = lax.axis_index("c")
    sid = lax.axis_index("s")
    wid = cid * NS + sid

    pltpu.sync_copy(dstx.at[wid], dst_v)

    z16 = jnp.zeros((16,), jnp.float32)
    o16 = jnp.ones((16,), jnp.float32)

    def zdrow(i, _):
        zd_v[i, pl.ds(0, 16)] = z16
        return 0
    lax.fori_loop(0, RPT, zdrow, 0)

    def orow(i, _):
        ones_v[i, pl.ds(0, 16)] = o16
        return 0
    lax.fori_loop(0, C, orow, 0)

    pltpu.sync_copy(zd_v, deg_sh.at[pl.ds(sid * RPT, RPT)])
    plsc.subcore_barrier()

    def step(j, _):
        pltpu.sync_copy(ones_v, deg_sh.at[dst_v.at[j]], add=True)
        return 0
    lax.fori_loop(0, CH, step, 0)
    plsc.subcore_barrier()

    sl = pl.ds(sid * RPT, RPT)
    pltpu.sync_copy(deg_sh.at[sl], deg.at[cid, sl])


_sc_deg = pl.kernel(
    _sc_deg_body,
    out_type=[jax.ShapeDtypeStruct((NC, NPAD, 16), jnp.float32)],
    mesh=plsc.VectorSubcoreMesh(core_axis_name="c", subcore_axis_name="s"),
    scratch_types=[
        pltpu.VMEM((CH, C), jnp.int32),
        pltpu.VMEM((RPT, 16), jnp.float32),
        pltpu.VMEM((C, 16), jnp.float32),
        pltpu.VMEM_SHARED((NPAD, 16), jnp.float32),
    ],
    compiler_params=pltpu.CompilerParams(use_tc_tiling_on_sc=False),
)


def _mm_body(x_ref, w_ref, ya, yb, r):
    yw = jnp.dot(x_ref[...], w_ref[...], preferred_element_type=jnp.float32)
    ya[...] = yw[:, :H]
    yb[...] = yw[:, H:D]
    r[...] = yw[:, D:]


def _matmul(x, w):
    return pl.pallas_call(
        _mm_body,
        grid=(N // BN,),
        in_specs=[pl.BlockSpec((BN, D), lambda i: (i, 0)),
                  pl.BlockSpec((D, 2 * D), lambda i: (0, 0))],
        out_specs=[pl.BlockSpec((BN, H), lambda i: (i, 0)),
                   pl.BlockSpec((BN, H), lambda i: (i, 0)),
                   pl.BlockSpec((BN, D), lambda i: (i, 0))],
        out_shape=[jax.ShapeDtypeStruct((N, H), jnp.float32),
                   jax.ShapeDtypeStruct((N, H), jnp.float32),
                   jax.ShapeDtypeStruct((N, D), jnp.float32)],
    )(x, w)


def _comb_mm_body(p0a, p0b, p1a, p1b, d0, d1, r1, b, w, ya, yb, r2):
    deg = jnp.maximum(d0[...] + d1[...], 1.0)[:, 0:1]
    ps = jnp.concatenate([p0a[...] + p1a[...], p0b[...] + p1b[...]], axis=1)
    hh = jnp.maximum(ps / deg + b[...] + r1[...], 0.0)
    yw = jnp.dot(hh, w[...], preferred_element_type=jnp.float32)
    ya[...] = yw[:, :H]
    yb[...] = yw[:, H:D]
    r2[...] = yw[:, D:]


def _combine_mm(p0a, p0b, p1a, p1b, d0, d1, r1, b, w):
    return pl.pallas_call(
        _comb_mm_body,
        grid=(N // BN,),
        in_specs=[pl.BlockSpec((BN, H), lambda i: (i, 0)),
                  pl.BlockSpec((BN, H), lambda i: (i, 0)),
                  pl.BlockSpec((BN, H), lambda i: (i, 0)),
                  pl.BlockSpec((BN, H), lambda i: (i, 0)),
                  pl.BlockSpec((BN, 16), lambda i: (i, 0)),
                  pl.BlockSpec((BN, 16), lambda i: (i, 0)),
                  pl.BlockSpec((BN, D), lambda i: (i, 0)),
                  pl.BlockSpec((1, D), lambda i: (0, 0)),
                  pl.BlockSpec((D, 2 * D), lambda i: (0, 0))],
        out_specs=[pl.BlockSpec((BN, H), lambda i: (i, 0)),
                   pl.BlockSpec((BN, H), lambda i: (i, 0)),
                   pl.BlockSpec((BN, D), lambda i: (i, 0))],
        out_shape=[jax.ShapeDtypeStruct((N, H), jnp.float32),
                   jax.ShapeDtypeStruct((N, H), jnp.float32),
                   jax.ShapeDtypeStruct((N, D), jnp.float32)],
    )(p0a, p0b, p1a, p1b, d0, d1, r1, b, w)


def _comb_pool_body(p0a, p0b, p1a, p1b, d0, d1, r2, b, wo, bo, out, acc):
    i = pl.program_id(0)
    deg = jnp.maximum(d0[...] + d1[...], 1.0)[:, 0:1]
    ps = jnp.concatenate([p0a[...] + p1a[...], p0b[...] + p1b[...]], axis=1)
    hh = jnp.maximum(ps / deg + b[...] + r2[...], 0.0)
    s = jnp.sum(hh, axis=0, keepdims=True)

    @pl.when(i == 0)
    def _():
        acc[...] = jnp.zeros_like(acc)

    acc[0:1, :] += s
    out[...] = (jnp.dot(acc[0:1, :] * (1.0 / N), wo[...],
                        preferred_element_type=jnp.float32) + bo[...])


def _combine_pool(p0a, p0b, p1a, p1b, d0, d1, r2, b, wo, bo):
    return pl.pallas_call(
        _comb_pool_body,
        grid=(N // BN,),
        in_specs=[pl.BlockSpec((BN, H), lambda i: (i, 0)),
                  pl.BlockSpec((BN, H), lambda i: (i, 0)),
                  pl.BlockSpec((BN, H), lambda i: (i, 0)),
                  pl.BlockSpec((BN, H), lambda i: (i, 0)),
                  pl.BlockSpec((BN, 16), lambda i: (i, 0)),
                  pl.BlockSpec((BN, 16), lambda i: (i, 0)),
                  pl.BlockSpec((BN, D), lambda i: (i, 0)),
                  pl.BlockSpec((1, D), lambda i: (0, 0)),
                  pl.BlockSpec((D, O), lambda i: (0, 0)),
                  pl.BlockSpec((1, O), lambda i: (0, 0))],
        out_specs=pl.BlockSpec((1, O), lambda i: (0, 0)),
        out_shape=jax.ShapeDtypeStruct((1, O), jnp.float32),
        scratch_shapes=[pltpu.VMEM((8, D), jnp.float32)],
    )(p0a, p0b, p1a, p1b, d0, d1, r2, b, wo, bo)


def kernel(x, edge_index, W1l, b1, W1r, W2l, b2, W2r, Wo, bo):
    src = edge_index[0].astype(jnp.int32)
    dst = edge_index[1].astype(jnp.int32)
    pad = EPAD - E
    srcx = jnp.concatenate([src, jnp.arange(pad, dtype=jnp.int32)]).reshape(NW, CH, C)
    dstx = jnp.concatenate([dst, jnp.full((pad,), N, jnp.int32)]).reshape(NW, CH, C)

    y1a, y1b, r1 = _matmul(x, jnp.concatenate([W1l, W1r], axis=1))
    (degs,) = _sc_deg(dstx)
    (parts1,) = _sc_segsum(y1a, y1b, srcx, dstx)
    y2a, y2b, r2 = _combine_mm(parts1[0, 0], parts1[0, 1],
                               parts1[1, 0], parts1[1, 1],
                               degs[0], degs[1],
                               r1, b1.reshape(1, D),
                               jnp.concatenate([W2l, W2r], axis=1))
    (parts2,) = _sc_segsum(y2a, y2b, srcx, dstx)
    return _combine_pool(parts2[0, 0], parts2[0, 1],
                         parts2[1, 0], parts2[1, 1],
                         degs[0], degs[1], r2,
                         b2.reshape(1, D), Wo, bo.reshape(1, O))


# speedup vs baseline: 2.6972x; 1.0968x over previous
import jax
import jax.numpy as jnp
from jax import lax
from jax.experimental import pallas as pl
from jax.experimental.pallas import tpu as pltpu
from jax.experimental.pallas import tpu_sc as plsc

N = 10000
E = 320000
D = 128
H = D // 2
O = 16

NC = 2
NS = 16
NW = NC * NS
C = 512
CH = 20
EW = C * CH
EPAD = NW * EW
NPAD = 10240
RPT = NPAD // NS
ZB = 128
BN = 1000


def _sc_segsum_body(ya, yb, srcx, dstx, part,
                    src_v, dst_v, rows_v, zb_v, acc_sh, *sems):
    cid = lax.axis_index("c")
    sid = lax.axis_index("s")
    wid = cid * NS + sid

    pltpu.sync_copy(srcx.at[wid], src_v)
    pltpu.sync_copy(dstx.at[wid], dst_v)

    z16 = jnp.zeros((16,), jnp.float32)

    def zrow(i, _):
        for k in range(H // 16):
            zb_v[i, pl.ds(k * 16, 16)] = z16
        return 0
    lax.fori_loop(0, ZB, zrow, 0)

    for h, y in ((0, ya), (1, yb)):
        for k in range(RPT // ZB):
            pltpu.sync_copy(zb_v, acc_sh.at[pl.ds(sid * RPT + k * ZB, ZB)])
        plsc.subcore_barrier()

        def step(j, _):
            pltpu.sync_copy(y.at[src_v.at[j]], rows_v)
            pltpu.sync_copy(rows_v, acc_sh.at[dst_v.at[j]], add=True)
            return 0
        lax.fori_loop(0, CH, step, 0)
        plsc.subcore_barrier()

        sl = pl.ds(sid * RPT, RPT)
        pltpu.sync_copy(acc_sh.at[sl], part.at[cid, h, sl])
        plsc.subcore_barrier()


_sc_segsum = pl.kernel(
    _sc_segsum_body,
    out_type=[jax.ShapeDtypeStruct((NC, 2, NPAD, H), jnp.float32)],
    mesh=plsc.VectorSubcoreMesh(core_axis_name="c", subcore_axis_name="s"),
    scratch_types=[
        pltpu.VMEM((CH, C), jnp.int32),
        pltpu.VMEM((CH, C), jnp.int32),
        pltpu.VMEM((C, H), jnp.float32),
        pltpu.VMEM((ZB, H), jnp.float32),
        pltpu.VMEM_SHARED((NPAD, H), jnp.float32),
    ],
    compiler_params=pltpu.CompilerParams(use_tc_tiling_on_sc=False),
)


def _sc_deg_body(dstx, deg, dst_v, zd_v, ones_v, deg_sh):
    cid = lax.axis_index("c")
    sid = lax.axis_index("s")
    wid = cid * NS + sid

    pltpu.sync_copy(dstx.at[wid], dst_v)

    z16 = jnp.zeros((16,), jnp.float32)
    o16 = jnp.ones((16,), jnp.float32)

    def zdrow(i, _):
        zd_v[i, pl.ds(0, 16)] = z16
        return 0
    lax.fori_loop(0, RPT, zdrow, 0)

    def orow(i, _):
        ones_v[i, pl.ds(0, 16)] = o16
        return 0
    lax.fori_loop(0, C, orow, 0)

    pltpu.sync_copy(zd_v, deg_sh.at[pl.ds(sid * RPT, RPT)])
    plsc.subcore_barrier()

    def step(j, _):
        pltpu.sync_copy(ones_v, deg_sh.at[dst_v.at[j]], add=True)
        return 0
    lax.fori_loop(0, CH, step, 0)
    plsc.subcore_barrier()

    sl = pl.ds(sid * RPT, RPT)
    pltpu.sync_copy(deg_sh.at[sl], deg.at[cid, sl])


_sc_deg = pl.kernel(
    _sc_deg_body,
    out_type=[jax.ShapeDtypeStruct((NC, NPAD, 16), jnp.float32)],
    mesh=plsc.VectorSubcoreMesh(core_axis_name="c", subcore_axis_name="s"),
    scratch_types=[
        pltpu.VMEM((CH, C), jnp.int32),
        pltpu.VMEM((RPT, 16), jnp.float32),
        pltpu.VMEM((C, 16), jnp.float32),
        pltpu.VMEM_SHARED((NPAD, 16), jnp.float32),
    ],
    compiler_params=pltpu.CompilerParams(use_tc_tiling_on_sc=False),
)


def _mm_body(x_ref, w_ref, ya, yb, r):
    yw = jnp.dot(x_ref[...], w_ref[...], preferred_element_type=jnp.float32)
    ya[...] = yw[:, :H]
    yb[...] = yw[:, H:D]
    r[...] = yw[:, D:]


def _matmul(x, w):
    return pl.pallas_call(
        _mm_body,
        grid=(N // BN,),
        in_specs=[pl.BlockSpec((BN, D), lambda i: (i, 0)),
                  pl.BlockSpec((D, 2 * D), lambda i: (0, 0))],
        out_specs=[pl.BlockSpec((BN, H), lambda i: (i, 0)),
                   pl.BlockSpec((BN, H), lambda i: (i, 0)),
                   pl.BlockSpec((BN, D), lambda i: (i, 0))],
        out_shape=[jax.ShapeDtypeStruct((N, H), jnp.float32),
                   jax.ShapeDtypeStruct((N, H), jnp.float32),
                   jax.ShapeDtypeStruct((N, D), jnp.float32)],
    )(x, w)


def _comb_mm_body(p0a, p0b, p1a, p1b, d0, d1, r1, b, w, ya, yb, r2):
    deg = jnp.maximum(d0[...] + d1[...], 1.0)[:, 0:1]
    ps = jnp.concatenate([p0a[...] + p1a[...], p0b[...] + p1b[...]], axis=1)
    hh = jnp.maximum(ps / deg + b[...] + r1[...], 0.0)
    yw = jnp.dot(hh, w[...], preferred_element_type=jnp.float32)
    ya[...] = yw[:, :H]
    yb[...] = yw[:, H:D]
    r2[...] = yw[:, D:]


def _combine_mm(p0a, p0b, p1a, p1b, d0, d1, r1, b, w):
    return pl.pallas_call(
        _comb_mm_body,
        grid=(N // BN,),
        in_specs=[pl.BlockSpec((BN, H), lambda i: (i, 0)),
                  pl.BlockSpec((BN, H), lambda i: (i, 0)),
                  pl.BlockSpec((BN, H), lambda i: (i, 0)),
                  pl.BlockSpec((BN, H), lambda i: (i, 0)),
                  pl.BlockSpec((BN, 16), lambda i: (i, 0)),
                  pl.BlockSpec((BN, 16), lambda i: (i, 0)),
                  pl.BlockSpec((BN, D), lambda i: (i, 0)),
                  pl.BlockSpec((1, D), lambda i: (0, 0)),
                  pl.BlockSpec((D, 2 * D), lambda i: (0, 0))],
        out_specs=[pl.BlockSpec((BN, H), lambda i: (i, 0)),
                   pl.BlockSpec((BN, H), lambda i: (i, 0)),
                   pl.BlockSpec((BN, D), lambda i: (i, 0))],
        out_shape=[jax.ShapeDtypeStruct((N, H), jnp.float32),
                   jax.ShapeDtypeStruct((N, H), jnp.float32),
                   jax.ShapeDtypeStruct((N, D), jnp.float32)],
    )(p0a, p0b, p1a, p1b, d0, d1, r1, b, w)


def _comb_pool_body(p0a, p0b, p1a, p1b, d0, d1, r2, b, wo, bo, out, acc):
    i = pl.program_id(0)
    deg = jnp.maximum(d0[...] + d1[...], 1.0)[:, 0:1]
    ps = jnp.concatenate([p0a[...] + p1a[...], p0b[...] + p1b[...]], axis=1)
    hh = jnp.maximum(ps / deg + b[...] + r2[...], 0.0)
    s = jnp.sum(hh, axis=0, keepdims=True)

    @pl.when(i == 0)
    def _():
        acc[...] = jnp.zeros_like(acc)

    acc[0:1, :] += s
    out[...] = (jnp.dot(acc[0:1, :] * (1.0 / N), wo[...],
                        preferred_element_type=jnp.float32) + bo[...])


def _combine_pool(p0a, p0b, p1a, p1b, d0, d1, r2, b, wo, bo):
    return pl.pallas_call(
        _comb_pool_body,
        grid=(N // BN,),
        in_specs=[pl.BlockSpec((BN, H), lambda i: (i, 0)),
                  pl.BlockSpec((BN, H), lambda i: (i, 0)),
                  pl.BlockSpec((BN, H), lambda i: (i, 0)),
                  pl.BlockSpec((BN, H), lambda i: (i, 0)),
                  pl.BlockSpec((BN, 16), lambda i: (i, 0)),
                  pl.BlockSpec((BN, 16), lambda i: (i, 0)),
                  pl.BlockSpec((BN, D), lambda i: (i, 0)),
                  pl.BlockSpec((1, D), lambda i: (0, 0)),
                  pl.BlockSpec((D, O), lambda i: (0, 0)),
                  pl.BlockSpec((1, O), lambda i: (0, 0))],
        out_specs=pl.BlockSpec((1, O), lambda i: (0, 0)),
        out_shape=jax.ShapeDtypeStruct((1, O), jnp.float32),
        scratch_shapes=[pltpu.VMEM((8, D), jnp.float32)],
    )(p0a, p0b, p1a, p1b, d0, d1, r2, b, wo, bo)


def kernel(x, edge_index, W1l, b1, W1r, W2l, b2, W2r, Wo, bo):
    src = edge_index[0].astype(jnp.int32)
    dst = edge_index[1].astype(jnp.int32)
    pad = EPAD - E
    srcx = jnp.concatenate([src, jnp.arange(pad, dtype=jnp.int32)]).reshape(NW, CH, C)
    dstx = jnp.concatenate([dst, jnp.full((pad,), N, jnp.int32)]).reshape(NW, CH, C)

    y1a, y1b, r1 = _matmul(x, jnp.concatenate([W1l, W1r], axis=1))
    (degs,) = _sc_deg(dstx)
    (parts1,) = _sc_segsum(y1a, y1b, srcx, dstx)
    y2a, y2b, r2 = _combine_mm(parts1[0, 0], parts1[0, 1],
                               parts1[1, 0], parts1[1, 1],
                               degs[0], degs[1],
                               r1, b1.reshape(1, D),
                               jnp.concatenate([W2l, W2r], axis=1))
    (parts2,) = _sc_segsum(y2a, y2b, srcx, dstx)
    return _combine_pool(parts2[0, 0], parts2[0, 1],
                         parts2[1, 0], parts2[1, 1],
                         degs[0], degs[1], r2,
                         b2.reshape(1, D), Wo, bo.reshape(1, O))


# speedup vs baseline: 3.1817x; 1.1796x over previous
import jax
import jax.numpy as jnp
from jax import lax
from jax.experimental import pallas as pl
from jax.experimental.pallas import tpu as pltpu
from jax.experimental.pallas import tpu_sc as plsc

N = 10000
E = 320000
D = 128
H = D // 2
O = 16

NC = 2
NS = 16
NW = NC * NS
C = 256
CH = 40
NB = 2
EW = C * CH
EPAD = NW * EW
NPAD = 10240
RPT = NPAD // NS
ZB = 128
BN = 1000


def _sc_segsum_body(ya, yb, srcx, dstx, part,
                    src_v, dst_v, rows_v, zb_v, acc_sh, *sems):
    cid = lax.axis_index("c")
    sid = lax.axis_index("s")
    wid = cid * NS + sid
    sem_g, sem_s = sems

    pltpu.sync_copy(srcx.at[wid], src_v)
    pltpu.sync_copy(dstx.at[wid], dst_v)

    z16 = jnp.zeros((16,), jnp.float32)

    def zrow(i, _):
        for k in range(H // 16):
            zb_v[i, pl.ds(k * 16, 16)] = z16
        return 0
    lax.fori_loop(0, ZB, zrow, 0)

    for h, y in ((0, ya), (1, yb)):
        for k in range(RPT // ZB):
            pltpu.sync_copy(zb_v, acc_sh.at[pl.ds(sid * RPT + k * ZB, ZB)])
        plsc.subcore_barrier()

        for b in range(NB):
            pltpu.async_copy(y.at[src_v.at[b]], rows_v.at[b], sem_g)

        def group(g, _):
            j0 = g * NB
            for b in range(NB):
                j = j0 + b
                pltpu.make_async_copy(
                    y.at[src_v.at[j]], rows_v.at[b], sem_g).wait()
                pltpu.async_copy(
                    rows_v.at[b], acc_sh.at[dst_v.at[j]], sem_s, add=True)
            for b in range(NB):
                j = j0 + b
                jn = j + NB
                pltpu.make_async_copy(
                    rows_v.at[b], acc_sh.at[dst_v.at[j]], sem_s).wait()

                @pl.when(jn < CH)
                def _():
                    pltpu.async_copy(y.at[src_v.at[jn]], rows_v.at[b], sem_g)
            return 0
        lax.fori_loop(0, CH // NB, group, 0)
        plsc.subcore_barrier()

        sl = pl.ds(sid * RPT, RPT)
        pltpu.sync_copy(acc_sh.at[sl], part.at[cid, h, sl])
        plsc.subcore_barrier()


_sc_segsum = pl.kernel(
    _sc_segsum_body,
    out_type=[jax.ShapeDtypeStruct((NC, 2, NPAD, H), jnp.float32)],
    mesh=plsc.VectorSubcoreMesh(core_axis_name="c", subcore_axis_name="s"),
    scratch_types=[
        pltpu.VMEM((CH, C), jnp.int32),
        pltpu.VMEM((CH, C), jnp.int32),
        pltpu.VMEM((NB, C, H), jnp.float32),
        pltpu.VMEM((ZB, H), jnp.float32),
        pltpu.VMEM_SHARED((NPAD, H), jnp.float32),
    ] + [pltpu.SemaphoreType.DMA] * 2,
    compiler_params=pltpu.CompilerParams(use_tc_tiling_on_sc=False),
)


def _sc_deg_body(dstx, deg, dst_v, zd_v, ones_v, deg_sh):
    cid = lax.axis_index("c")
    sid = lax.axis_index("s")
    wid = cid * NS + sid

    pltpu.sync_copy(dstx.at[wid], dst_v)

    z16 = jnp.zeros((16,), jnp.float32)
    o16 = jnp.ones((16,), jnp.float32)

    def zdrow(i, _):
        zd_v[i, pl.ds(0, 16)] = z16
        return 0
    lax.fori_loop(0, RPT, zdrow, 0)

    def orow(i, _):
        ones_v[i, pl.ds(0, 16)] = o16
        return 0
    lax.fori_loop(0, C, orow, 0)

    pltpu.sync_copy(zd_v, deg_sh.at[pl.ds(sid * RPT, RPT)])
    plsc.subcore_barrier()

    def step(j, _):
        pltpu.sync_copy(ones_v, deg_sh.at[dst_v.at[j]], add=True)
        return 0
    lax.fori_loop(0, CH, step, 0)
    plsc.subcore_barrier()

    sl = pl.ds(sid * RPT, RPT)
    pltpu.sync_copy(deg_sh.at[sl], deg.at[cid, sl])


_sc_deg = pl.kernel(
    _sc_deg_body,
    out_type=[jax.ShapeDtypeStruct((NC, NPAD, 16), jnp.float32)],
    mesh=plsc.VectorSubcoreMesh(core_axis_name="c", subcore_axis_name="s"),
    scratch_types=[
        pltpu.VMEM((CH, C), jnp.int32),
        pltpu.VMEM((RPT, 16), jnp.float32),
        pltpu.VMEM((C, 16), jnp.float32),
        pltpu.VMEM_SHARED((NPAD, 16), jnp.float32),
    ],
    compiler_params=pltpu.CompilerParams(use_tc_tiling_on_sc=False),
)


def _mm_body(x_ref, w_ref, ya, yb, r):
    yw = jnp.dot(x_ref[...], w_ref[...], preferred_element_type=jnp.float32)
    ya[...] = yw[:, :H]
    yb[...] = yw[:, H:D]
    r[...] = yw[:, D:]


def _matmul(x, w):
    return pl.pallas_call(
        _mm_body,
        grid=(N // BN,),
        in_specs=[pl.BlockSpec((BN, D), lambda i: (i, 0)),
                  pl.BlockSpec((D, 2 * D), lambda i: (0, 0))],
        out_specs=[pl.BlockSpec((BN, H), lambda i: (i, 0)),
                   pl.BlockSpec((BN, H), lambda i: (i, 0)),
                   pl.BlockSpec((BN, D), lambda i: (i, 0))],
        out_shape=[jax.ShapeDtypeStruct((N, H), jnp.float32),
                   jax.ShapeDtypeStruct((N, H), jnp.float32),
                   jax.ShapeDtypeStruct((N, D), jnp.float32)],
    )(x, w)


def _comb_mm_body(p0a, p0b, p1a, p1b, d0, d1, r1, b, w, ya, yb, r2):
    deg = jnp.maximum(d0[...] + d1[...], 1.0)[:, 0:1]
    ps = jnp.concatenate([p0a[...] + p1a[...], p0b[...] + p1b[...]], axis=1)
    hh = jnp.maximum(ps / deg + b[...] + r1[...], 0.0)
    yw = jnp.dot(hh, w[...], preferred_element_type=jnp.float32)
    ya[...] = yw[:, :H]
    yb[...] = yw[:, H:D]
    r2[...] = yw[:, D:]


def _combine_mm(p0a, p0b, p1a, p1b, d0, d1, r1, b, w):
    return pl.pallas_call(
        _comb_mm_body,
        grid=(N // BN,),
        in_specs=[pl.BlockSpec((BN, H), lambda i: (i, 0)),
                  pl.BlockSpec((BN, H), lambda i: (i, 0)),
                  pl.BlockSpec((BN, H), lambda i: (i, 0)),
                  pl.BlockSpec((BN, H), lambda i: (i, 0)),
                  pl.BlockSpec((BN, 16), lambda i: (i, 0)),
                  pl.BlockSpec((BN, 16), lambda i: (i, 0)),
                  pl.BlockSpec((BN, D), lambda i: (i, 0)),
                  pl.BlockSpec((1, D), lambda i: (0, 0)),
                  pl.BlockSpec((D, 2 * D), lambda i: (0, 0))],
        out_specs=[pl.BlockSpec((BN, H), lambda i: (i, 0)),
                   pl.BlockSpec((BN, H), lambda i: (i, 0)),
                   pl.BlockSpec((BN, D), lambda i: (i, 0))],
        out_shape=[jax.ShapeDtypeStruct((N, H), jnp.float32),
                   jax.ShapeDtypeStruct((N, H), jnp.float32),
                   jax.ShapeDtypeStruct((N, D), jnp.float32)],
    )(p0a, p0b, p1a, p1b, d0, d1, r1, b, w)


def _comb_pool_body(p0a, p0b, p1a, p1b, d0, d1, r2, b, wo, bo, out, acc):
    i = pl.program_id(0)
    deg = jnp.maximum(d0[...] + d1[...], 1.0)[:, 0:1]
    ps = jnp.concatenate([p0a[...] + p1a[...], p0b[...] + p1b[...]], axis=1)
    hh = jnp.maximum(ps / deg + b[...] + r2[...], 0.0)
    s = jnp.sum(hh, axis=0, keepdims=True)

    @pl.when(i == 0)
    def _():
        acc[...] = jnp.zeros_like(acc)

    acc[0:1, :] += s
    out[...] = (jnp.dot(acc[0:1, :] * (1.0 / N), wo[...],
                        preferred_element_type=jnp.float32) + bo[...])


def _combine_pool(p0a, p0b, p1a, p1b, d0, d1, r2, b, wo, bo):
    return pl.pallas_call(
        _comb_pool_body,
        grid=(N // BN,),
        in_specs=[pl.BlockSpec((BN, H), lambda i: (i, 0)),
                  pl.BlockSpec((BN, H), lambda i: (i, 0)),
                  pl.BlockSpec((BN, H), lambda i: (i, 0)),
                  pl.BlockSpec((BN, H), lambda i: (i, 0)),
                  pl.BlockSpec((BN, 16), lambda i: (i, 0)),
                  pl.BlockSpec((BN, 16), lambda i: (i, 0)),
                  pl.BlockSpec((BN, D), lambda i: (i, 0)),
                  pl.BlockSpec((1, D), lambda i: (0, 0)),
                  pl.BlockSpec((D, O), lambda i: (0, 0)),
                  pl.BlockSpec((1, O), lambda i: (0, 0))],
        out_specs=pl.BlockSpec((1, O), lambda i: (0, 0)),
        out_shape=jax.ShapeDtypeStruct((1, O), jnp.float32),
        scratch_shapes=[pltpu.VMEM((8, D), jnp.float32)],
    )(p0a, p0b, p1a, p1b, d0, d1, r2, b, wo, bo)


def kernel(x, edge_index, W1l, b1, W1r, W2l, b2, W2r, Wo, bo):
    src = edge_index[0].astype(jnp.int32)
    dst = edge_index[1].astype(jnp.int32)
    pad = EPAD - E
    srcx = jnp.concatenate([src, jnp.arange(pad, dtype=jnp.int32)]).reshape(NW, CH, C)
    dstx = jnp.concatenate([dst, jnp.full((pad,), N, jnp.int32)]).reshape(NW, CH, C)

    y1a, y1b, r1 = _matmul(x, jnp.concatenate([W1l, W1r], axis=1))
    (degs,) = _sc_deg(dstx)
    (parts1,) = _sc_segsum(y1a, y1b, srcx, dstx)
    y2a, y2b, r2 = _combine_mm(parts1[0, 0], parts1[0, 1],
                               parts1[1, 0], parts1[1, 1],
                               degs[0], degs[1],
                               r1, b1.reshape(1, D),
                               jnp.concatenate([W2l, W2r], axis=1))
    (parts2,) = _sc_segsum(y2a, y2b, srcx, dstx)
    return _combine_pool(parts2[0, 0], parts2[0, 1],
                         parts2[1, 0], parts2[1, 1],
                         degs[0], degs[1], r2,
                         b2.reshape(1, D), Wo, bo.reshape(1, O))


# speedup vs baseline: 3.2122x; 1.0096x over previous
import jax
import jax.numpy as jnp
from jax import lax
from jax.experimental import pallas as pl
from jax.experimental.pallas import tpu as pltpu
from jax.experimental.pallas import tpu_sc as plsc

N = 10000
E = 320000
D = 128
H = D // 2
O = 16

NC = 2
NS = 16
NW = NC * NS
C = 320
CH = 32
NB = 2
EW = C * CH
EPAD = NW * EW
NPAD = 10240
RPT = NPAD // NS
ZB = 128
BN = 1000


def _sc_segsum_body(ya, yb, srcx, dstx, part,
                    src_v, dst_v, rows_v, zb_v, acc_sh, *sems):
    cid = lax.axis_index("c")
    sid = lax.axis_index("s")
    wid = cid * NS + sid
    sem_g, sem_s = sems

    pltpu.sync_copy(srcx.at[wid], src_v)
    pltpu.sync_copy(dstx.at[wid], dst_v)

    z16 = jnp.zeros((16,), jnp.float32)

    def zrow(i, _):
        for k in range(H // 16):
            zb_v[i, pl.ds(k * 16, 16)] = z16
        return 0
    lax.fori_loop(0, ZB, zrow, 0)

    for h, y in ((0, ya), (1, yb)):
        for k in range(RPT // ZB):
            pltpu.sync_copy(zb_v, acc_sh.at[pl.ds(sid * RPT + k * ZB, ZB)])
        plsc.subcore_barrier()

        for b in range(NB):
            pltpu.async_copy(y.at[src_v.at[b]], rows_v.at[b], sem_g)

        def group(g, _):
            j0 = g * NB
            for b in range(NB):
                j = j0 + b
                pltpu.make_async_copy(
                    y.at[src_v.at[j]], rows_v.at[b], sem_g).wait()
                pltpu.async_copy(
                    rows_v.at[b], acc_sh.at[dst_v.at[j]], sem_s, add=True)
            for b in range(NB):
                j = j0 + b
                jn = j + NB
                pltpu.make_async_copy(
                    rows_v.at[b], acc_sh.at[dst_v.at[j]], sem_s).wait()

                @pl.when(jn < CH)
                def _():
                    pltpu.async_copy(y.at[src_v.at[jn]], rows_v.at[b], sem_g)
            return 0
        lax.fori_loop(0, CH // NB, group, 0)
        plsc.subcore_barrier()

        sl = pl.ds(sid * RPT, RPT)
        pltpu.sync_copy(acc_sh.at[sl], part.at[cid, h, sl])
        plsc.subcore_barrier()


_sc_segsum = pl.kernel(
    _sc_segsum_body,
    out_type=[jax.ShapeDtypeStruct((NC, 2, NPAD, H), jnp.float32)],
    mesh=plsc.VectorSubcoreMesh(core_axis_name="c", subcore_axis_name="s"),
    scratch_types=[
        pltpu.VMEM((CH, C), jnp.int32),
        pltpu.VMEM((CH, C), jnp.int32),
        pltpu.VMEM((NB, C, H), jnp.float32),
        pltpu.VMEM((ZB, H), jnp.float32),
        pltpu.VMEM_SHARED((NPAD, H), jnp.float32),
    ] + [pltpu.SemaphoreType.DMA] * 2,
    compiler_params=pltpu.CompilerParams(use_tc_tiling_on_sc=False),
)


def _sc_deg_body(dstx, deg, dst_v, zd_v, ones_v, deg_sh):
    cid = lax.axis_index("c")
    sid = lax.axis_index("s")
    wid = cid * NS + sid

    pltpu.sync_copy(dstx.at[wid], dst_v)

    z16 = jnp.zeros((16,), jnp.float32)
    o16 = jnp.ones((16,), jnp.float32)

    def zdrow(i, _):
        zd_v[i, pl.ds(0, 16)] = z16
        return 0
    lax.fori_loop(0, RPT, zdrow, 0)

    def orow(i, _):
        ones_v[i, pl.ds(0, 16)] = o16
        return 0
    lax.fori_loop(0, C, orow, 0)

    pltpu.sync_copy(zd_v, deg_sh.at[pl.ds(sid * RPT, RPT)])
    plsc.subcore_barrier()

    def step(j, _):
        pltpu.sync_copy(ones_v, deg_sh.at[dst_v.at[j]], add=True)
        return 0
    lax.fori_loop(0, CH, step, 0)
    plsc.subcore_barrier()

    sl = pl.ds(sid * RPT, RPT)
    pltpu.sync_copy(deg_sh.at[sl], deg.at[cid, sl])


_sc_deg = pl.kernel(
    _sc_deg_body,
    out_type=[jax.ShapeDtypeStruct((NC, NPAD, 16), jnp.float32)],
    mesh=plsc.VectorSubcoreMesh(core_axis_name="c", subcore_axis_name="s"),
    scratch_types=[
        pltpu.VMEM((CH, C), jnp.int32),
        pltpu.VMEM((RPT, 16), jnp.float32),
        pltpu.VMEM((C, 16), jnp.float32),
        pltpu.VMEM_SHARED((NPAD, 16), jnp.float32),
    ],
    compiler_params=pltpu.CompilerParams(use_tc_tiling_on_sc=False),
)


def _mm_body(x_ref, w_ref, ya, yb, r):
    yw = jnp.dot(x_ref[...], w_ref[...], preferred_element_type=jnp.float32)
    ya[...] = yw[:, :H]
    yb[...] = yw[:, H:D]
    r[...] = yw[:, D:]


def _matmul(x, w):
    return pl.pallas_call(
        _mm_body,
        grid=(N // BN,),
        in_specs=[pl.BlockSpec((BN, D), lambda i: (i, 0)),
                  pl.BlockSpec((D, 2 * D), lambda i: (0, 0))],
        out_specs=[pl.BlockSpec((BN, H), lambda i: (i, 0)),
                   pl.BlockSpec((BN, H), lambda i: (i, 0)),
                   pl.BlockSpec((BN, D), lambda i: (i, 0))],
        out_shape=[jax.ShapeDtypeStruct((N, H), jnp.float32),
                   jax.ShapeDtypeStruct((N, H), jnp.float32),
                   jax.ShapeDtypeStruct((N, D), jnp.float32)],
    )(x, w)


def _comb_mm_body(p0a, p0b, p1a, p1b, d0, d1, r1, b, w, ya, yb, r2):
    deg = jnp.maximum(d0[...] + d1[...], 1.0)[:, 0:1]
    ps = jnp.concatenate([p0a[...] + p1a[...], p0b[...] + p1b[...]], axis=1)
    hh = jnp.maximum(ps / deg + b[...] + r1[...], 0.0)
    yw = jnp.dot(hh, w[...], preferred_element_type=jnp.float32)
    ya[...] = yw[:, :H]
    yb[...] = yw[:, H:D]
    r2[...] = yw[:, D:]


def _combine_mm(p0a, p0b, p1a, p1b, d0, d1, r1, b, w):
    return pl.pallas_call(
        _comb_mm_body,
        grid=(N // BN,),
        in_specs=[pl.BlockSpec((BN, H), lambda i: (i, 0)),
                  pl.BlockSpec((BN, H), lambda i: (i, 0)),
                  pl.BlockSpec((BN, H), lambda i: (i, 0)),
                  pl.BlockSpec((BN, H), lambda i: (i, 0)),
                  pl.BlockSpec((BN, 16), lambda i: (i, 0)),
                  pl.BlockSpec((BN, 16), lambda i: (i, 0)),
                  pl.BlockSpec((BN, D), lambda i: (i, 0)),
                  pl.BlockSpec((1, D), lambda i: (0, 0)),
                  pl.BlockSpec((D, 2 * D), lambda i: (0, 0))],
        out_specs=[pl.BlockSpec((BN, H), lambda i: (i, 0)),
                   pl.BlockSpec((BN, H), lambda i: (i, 0)),
                   pl.BlockSpec((BN, D), lambda i: (i, 0))],
        out_shape=[jax.ShapeDtypeStruct((N, H), jnp.float32),
                   jax.ShapeDtypeStruct((N, H), jnp.float32),
                   jax.ShapeDtypeStruct((N, D), jnp.float32)],
    )(p0a, p0b, p1a, p1b, d0, d1, r1, b, w)


def _comb_pool_body(p0a, p0b, p1a, p1b, d0, d1, r2, b, wo, bo, out, acc):
    i = pl.program_id(0)
    deg = jnp.maximum(d0[...] + d1[...], 1.0)[:, 0:1]
    ps = jnp.concatenate([p0a[...] + p1a[...], p0b[...] + p1b[...]], axis=1)
    hh = jnp.maximum(ps / deg + b[...] + r2[...], 0.0)
    s = jnp.sum(hh, axis=0, keepdims=True)

    @pl.when(i == 0)
    def _():
        acc[...] = jnp.zeros_like(acc)

    acc[0:1, :] += s
    out[...] = (jnp.dot(acc[0:1, :] * (1.0 / N), wo[...],
                        preferred_element_type=jnp.float32) + bo[...])


def _combine_pool(p0a, p0b, p1a, p1b, d0, d1, r2, b, wo, bo):
    return pl.pallas_call(
        _comb_pool_body,
        grid=(N // BN,),
        in_specs=[pl.BlockSpec((BN, H), lambda i: (i, 0)),
                  pl.BlockSpec((BN, H), lambda i: (i, 0)),
                  pl.BlockSpec((BN, H), lambda i: (i, 0)),
                  pl.BlockSpec((BN, H), lambda i: (i, 0)),
                  pl.BlockSpec((BN, 16), lambda i: (i, 0)),
                  pl.BlockSpec((BN, 16), lambda i: (i, 0)),
                  pl.BlockSpec((BN, D), lambda i: (i, 0)),
                  pl.BlockSpec((1, D), lambda i: (0, 0)),
                  pl.BlockSpec((D, O), lambda i: (0, 0)),
                  pl.BlockSpec((1, O), lambda i: (0, 0))],
        out_specs=pl.BlockSpec((1, O), lambda i: (0, 0)),
        out_shape=jax.ShapeDtypeStruct((1, O), jnp.float32),
        scratch_shapes=[pltpu.VMEM((8, D), jnp.float32)],
    )(p0a, p0b, p1a, p1b, d0, d1, r2, b, wo, bo)


def kernel(x, edge_index, W1l, b1, W1r, W2l, b2, W2r, Wo, bo):
    src = edge_index[0].astype(jnp.int32)
    dst = edge_index[1].astype(jnp.int32)
    pad = EPAD - E
    srcx = jnp.concatenate([src, jnp.arange(pad, dtype=jnp.int32)]).reshape(NW, CH, C)
    dstx = jnp.concatenate([dst, jnp.full((pad,), N, jnp.int32)]).reshape(NW, CH, C)

    y1a, y1b, r1 = _matmul(x, jnp.concatenate([W1l, W1r], axis=1))
    (degs,) = _sc_deg(dstx)
    (parts1,) = _sc_segsum(y1a, y1b, srcx, dstx)
    y2a, y2b, r2 = _combine_mm(parts1[0, 0], parts1[0, 1],
                               parts1[1, 0], parts1[1, 1],
                               degs[0], degs[1],
                               r1, b1.reshape(1, D),
                               jnp.concatenate([W2l, W2r], axis=1))
    (parts2,) = _sc_segsum(y2a, y2b, srcx, dstx)
    return _combine_pool(parts2[0, 0], parts2[0, 1],
                         parts2[1, 0], parts2[1, 1],
                         degs[0], degs[1], r2,
                         b2.reshape(1, D), Wo, bo.reshape(1, O))


# speedup vs baseline: 3.2257x; 1.0042x over previous
import jax
import jax.numpy as jnp
from jax import lax
from jax.experimental import pallas as pl
from jax.experimental.pallas import tpu as pltpu
from jax.experimental.pallas import tpu_sc as plsc

N = 10000
E = 320000
D = 128
H = D // 2
O = 16

NC = 2
NS = 16
NW = NC * NS
C = 128
CH = 80
NB = 4
EW = C * CH
EPAD = NW * EW
NPAD = 10240
RPT = NPAD // NS
ZB = 128
BN = 1000


def _sc_segsum_body(ya, yb, srcx, dstx, part,
                    src_v, dst_v, rows_v, zb_v, acc_sh, *sems):
    cid = lax.axis_index("c")
    sid = lax.axis_index("s")
    wid = cid * NS + sid
    sem_g, sem_s = sems

    pltpu.sync_copy(srcx.at[wid], src_v)
    pltpu.sync_copy(dstx.at[wid], dst_v)

    z16 = jnp.zeros((16,), jnp.float32)

    def zrow(i, _):
        for k in range(H // 16):
            zb_v[i, pl.ds(k * 16, 16)] = z16
        return 0
    lax.fori_loop(0, ZB, zrow, 0)

    for h, y in ((0, ya), (1, yb)):
        for k in range(RPT // ZB):
            pltpu.sync_copy(zb_v, acc_sh.at[pl.ds(sid * RPT + k * ZB, ZB)])
        plsc.subcore_barrier()

        for b in range(NB):
            pltpu.async_copy(y.at[src_v.at[b]], rows_v.at[b], sem_g)

        def group(g, _):
            j0 = g * NB
            for b in range(NB):
                j = j0 + b
                pltpu.make_async_copy(
                    y.at[src_v.at[j]], rows_v.at[b], sem_g).wait()
                pltpu.async_copy(
                    rows_v.at[b], acc_sh.at[dst_v.at[j]], sem_s, add=True)
            for b in range(NB):
                j = j0 + b
                jn = j + NB
                pltpu.make_async_copy(
                    rows_v.at[b], acc_sh.at[dst_v.at[j]], sem_s).wait()

                @pl.when(jn < CH)
                def _():
                    pltpu.async_copy(y.at[src_v.at[jn]], rows_v.at[b], sem_g)
            return 0
        lax.fori_loop(0, CH // NB, group, 0)
        plsc.subcore_barrier()

        sl = pl.ds(sid * RPT, RPT)
        pltpu.sync_copy(acc_sh.at[sl], part.at[cid, h, sl])
        plsc.subcore_barrier()


_sc_segsum = pl.kernel(
    _sc_segsum_body,
    out_type=[jax.ShapeDtypeStruct((NC, 2, NPAD, H), jnp.float32)],
    mesh=plsc.VectorSubcoreMesh(core_axis_name="c", subcore_axis_name="s"),
    scratch_types=[
        pltpu.VMEM((CH, C), jnp.int32),
        pltpu.VMEM((CH, C), jnp.int32),
        pltpu.VMEM((NB, C, H), jnp.float32),
        pltpu.VMEM((ZB, H), jnp.float32),
        pltpu.VMEM_SHARED((NPAD, H), jnp.float32),
    ] + [pltpu.SemaphoreType.DMA] * 2,
    compiler_params=pltpu.CompilerParams(use_tc_tiling_on_sc=False),
)


def _sc_deg_body(dstx, deg, dst_v, zd_v, ones_v, deg_sh):
    cid = lax.axis_index("c")
    sid = lax.axis_index("s")
    wid = cid * NS + sid

    pltpu.sync_copy(dstx.at[wid], dst_v)

    z16 = jnp.zeros((16,), jnp.float32)
    o16 = jnp.ones((16,), jnp.float32)

    def zdrow(i, _):
        zd_v[i, pl.ds(0, 16)] = z16
        return 0
    lax.fori_loop(0, RPT, zdrow, 0)

    def orow(i, _):
        ones_v[i, pl.ds(0, 16)] = o16
        return 0
    lax.fori_loop(0, C, orow, 0)

    pltpu.sync_copy(zd_v, deg_sh.at[pl.ds(sid * RPT, RPT)])
    plsc.subcore_barrier()

    def step(j, _):
        pltpu.sync_copy(ones_v, deg_sh.at[dst_v.at[j]], add=True)
        return 0
    lax.fori_loop(0, CH, step, 0)
    plsc.subcore_barrier()

    sl = pl.ds(sid * RPT, RPT)
    pltpu.sync_copy(deg_sh.at[sl], deg.at[cid, sl])


_sc_deg = pl.kernel(
    _sc_deg_body,
    out_type=[jax.ShapeDtypeStruct((NC, NPAD, 16), jnp.float32)],
    mesh=plsc.VectorSubcoreMesh(core_axis_name="c", subcore_axis_name="s"),
    scratch_types=[
        pltpu.VMEM((CH, C), jnp.int32),
        pltpu.VMEM((RPT, 16), jnp.float32),
        pltpu.VMEM((C, 16), jnp.float32),
        pltpu.VMEM_SHARED((NPAD, 16), jnp.float32),
    ],
    compiler_params=pltpu.CompilerParams(use_tc_tiling_on_sc=False),
)


def _mm_body(x_ref, w_ref, ya, yb, r):
    yw = jnp.dot(x_ref[...], w_ref[...], preferred_element_type=jnp.float32)
    ya[...] = yw[:, :H]
    yb[...] = yw[:, H:D]
    r[...] = yw[:, D:]


def _matmul(x, w):
    return pl.pallas_call(
        _mm_body,
        grid=(N // BN,),
        in_specs=[pl.BlockSpec((BN, D), lambda i: (i, 0)),
                  pl.BlockSpec((D, 2 * D), lambda i: (0, 0))],
        out_specs=[pl.BlockSpec((BN, H), lambda i: (i, 0)),
                   pl.BlockSpec((BN, H), lambda i: (i, 0)),
                   pl.BlockSpec((BN, D), lambda i: (i, 0))],
        out_shape=[jax.ShapeDtypeStruct((N, H), jnp.float32),
                   jax.ShapeDtypeStruct((N, H), jnp.float32),
                   jax.ShapeDtypeStruct((N, D), jnp.float32)],
    )(x, w)


def _comb_mm_body(p0a, p0b, p1a, p1b, d0, d1, r1, b, w, ya, yb, r2):
    deg = jnp.maximum(d0[...] + d1[...], 1.0)[:, 0:1]
    ps = jnp.concatenate([p0a[...] + p1a[...], p0b[...] + p1b[...]], axis=1)
    hh = jnp.maximum(ps / deg + b[...] + r1[...], 0.0)
    yw = jnp.dot(hh, w[...], preferred_element_type=jnp.float32)
    ya[...] = yw[:, :H]
    yb[...] = yw[:, H:D]
    r2[...] = yw[:, D:]


def _combine_mm(p0a, p0b, p1a, p1b, d0, d1, r1, b, w):
    return pl.pallas_call(
        _comb_mm_body,
        grid=(N // BN,),
        in_specs=[pl.BlockSpec((BN, H), lambda i: (i, 0)),
                  pl.BlockSpec((BN, H), lambda i: (i, 0)),
                  pl.BlockSpec((BN, H), lambda i: (i, 0)),
                  pl.BlockSpec((BN, H), lambda i: (i, 0)),
                  pl.BlockSpec((BN, 16), lambda i: (i, 0)),
                  pl.BlockSpec((BN, 16), lambda i: (i, 0)),
                  pl.BlockSpec((BN, D), lambda i: (i, 0)),
                  pl.BlockSpec((1, D), lambda i: (0, 0)),
                  pl.BlockSpec((D, 2 * D), lambda i: (0, 0))],
        out_specs=[pl.BlockSpec((BN, H), lambda i: (i, 0)),
                   pl.BlockSpec((BN, H), lambda i: (i, 0)),
                   pl.BlockSpec((BN, D), lambda i: (i, 0))],
        out_shape=[jax.ShapeDtypeStruct((N, H), jnp.float32),
                   jax.ShapeDtypeStruct((N, H), jnp.float32),
                   jax.ShapeDtypeStruct((N, D), jnp.float32)],
    )(p0a, p0b, p1a, p1b, d0, d1, r1, b, w)


def _comb_pool_body(p0a, p0b, p1a, p1b, d0, d1, r2, b, wo, bo, out, acc):
    i = pl.program_id(0)
    deg = jnp.maximum(d0[...] + d1[...], 1.0)[:, 0:1]
    ps = jnp.concatenate([p0a[...] + p1a[...], p0b[...] + p1b[...]], axis=1)
    hh = jnp.maximum(ps / deg + b[...] + r2[...], 0.0)
    s = jnp.sum(hh, axis=0, keepdims=True)

    @pl.when(i == 0)
    def _():
        acc[...] = jnp.zeros_like(acc)

    acc[0:1, :] += s
    out[...] = (jnp.dot(acc[0:1, :] * (1.0 / N), wo[...],
                        preferred_element_type=jnp.float32) + bo[...])


def _combine_pool(p0a, p0b, p1a, p1b, d0, d1, r2, b, wo, bo):
    return pl.pallas_call(
        _comb_pool_body,
        grid=(N // BN,),
        in_specs=[pl.BlockSpec((BN, H), lambda i: (i, 0)),
                  pl.BlockSpec((BN, H), lambda i: (i, 0)),
                  pl.BlockSpec((BN, H), lambda i: (i, 0)),
                  pl.BlockSpec((BN, H), lambda i: (i, 0)),
                  pl.BlockSpec((BN, 16), lambda i: (i, 0)),
                  pl.BlockSpec((BN, 16), lambda i: (i, 0)),
                  pl.BlockSpec((BN, D), lambda i: (i, 0)),
                  pl.BlockSpec((1, D), lambda i: (0, 0)),
                  pl.BlockSpec((D, O), lambda i: (0, 0)),
                  pl.BlockSpec((1, O), lambda i: (0, 0))],
        out_specs=pl.BlockSpec((1, O), lambda i: (0, 0)),
        out_shape=jax.ShapeDtypeStruct((1, O), jnp.float32),
        scratch_shapes=[pltpu.VMEM((8, D), jnp.float32)],
    )(p0a, p0b, p1a, p1b, d0, d1, r2, b, wo, bo)


def kernel(x, edge_index, W1l, b1, W1r, W2l, b2, W2r, Wo, bo):
    src = edge_index[0].astype(jnp.int32)
    dst = edge_index[1].astype(jnp.int32)
    pad = EPAD - E
    srcx = jnp.concatenate([src, jnp.arange(pad, dtype=jnp.int32)]).reshape(NW, CH, C)
    dstx = jnp.concatenate([dst, jnp.full((pad,), N, jnp.int32)]).reshape(NW, CH, C)

    y1a, y1b, r1 = _matmul(x, jnp.concatenate([W1l, W1r], axis=1))
    (degs,) = _sc_deg(dstx)
    (parts1,) = _sc_segsum(y1a, y1b, srcx, dstx)
    y2a, y2b, r2 = _combine_mm(parts1[0, 0], parts1[0, 1],
                               parts1[1, 0], parts1[1, 1],
                               degs[0], degs[1],
                               r1, b1.reshape(1, D),
                               jnp.concatenate([W2l, W2r], axis=1))
    (parts2,) = _sc_segsum(y2a, y2b, srcx, dstx)
    return _combine_pool(parts2[0, 0], parts2[0, 1],
                         parts2[1, 0], parts2[1, 1],
                         degs[0], degs[1], r2,
                         b2.reshape(1, D), Wo, bo.reshape(1, O))


# speedup vs baseline: 3.5242x; 1.0925x over previous
import jax
import jax.numpy as jnp
from jax import lax
from jax.experimental import pallas as pl
from jax.experimental.pallas import tpu as pltpu
from jax.experimental.pallas import tpu_sc as plsc

N = 10000
E = 320000
D = 128
H = D // 2
O = 16

NC = 2
NS = 16
NW = NC * NS
C = 128
CH = 80
NB = 4
EW = C * CH
EPAD = NW * EW
NPAD = 10240
RPT = NPAD // NS
ZB = 128
BN = 1000


def _sc_segsum_body(ya, yb, srcx, dstx, part,
                    src_v, dst_v, rows_v, zb_v, acc_sh, *sems):
    cid = lax.axis_index("c")
    sid = lax.axis_index("s")
    wid = cid * NS + sid
    sem_g, sem_s = sems

    pltpu.sync_copy(srcx.at[wid], src_v)
    pltpu.sync_copy(dstx.at[wid], dst_v)

    z16 = jnp.zeros((16,), jnp.float32)

    def zrow(i, _):
        for k in range(H // 16):
            zb_v[i, pl.ds(k * 16, 16)] = z16
        return 0
    lax.fori_loop(0, ZB, zrow, 0)

    for h, y in ((0, ya), (1, yb)):
        for k in range(RPT // ZB):
            pltpu.sync_copy(zb_v, acc_sh.at[pl.ds(sid * RPT + k * ZB, ZB)])
        plsc.subcore_barrier()

        for b in range(NB):
            pltpu.async_copy(y.at[src_v.at[b]], rows_v.at[b], sem_g)

        def group(g, _):
            j0 = g * NB
            for b in range(NB):
                j = j0 + b
                pltpu.make_async_copy(
                    y.at[src_v.at[j]], rows_v.at[b], sem_g).wait()
                pltpu.async_copy(
                    rows_v.at[b], acc_sh.at[dst_v.at[j]], sem_s, add=True)
            for b in range(NB):
                j = j0 + b
                jn = j + NB
                pltpu.make_async_copy(
                    rows_v.at[b], acc_sh.at[dst_v.at[j]], sem_s).wait()

                @pl.when(jn < CH)
                def _():
                    pltpu.async_copy(y.at[src_v.at[jn]], rows_v.at[b], sem_g)
            return 0
        lax.fori_loop(0, CH // NB, group, 0)
        plsc.subcore_barrier()

        sl = pl.ds(sid * RPT, RPT)
        pltpu.sync_copy(acc_sh.at[sl], part.at[cid, sl, pl.ds(h * H, H)])
        plsc.subcore_barrier()


_sc_segsum = pl.kernel(
    _sc_segsum_body,
    out_type=[jax.ShapeDtypeStruct((NC, NPAD, D), jnp.float32)],
    mesh=plsc.VectorSubcoreMesh(core_axis_name="c", subcore_axis_name="s"),
    scratch_types=[
        pltpu.VMEM((CH, C), jnp.int32),
        pltpu.VMEM((CH, C), jnp.int32),
        pltpu.VMEM((NB, C, H), jnp.float32),
        pltpu.VMEM((ZB, H), jnp.float32),
        pltpu.VMEM_SHARED((NPAD, H), jnp.float32),
    ] + [pltpu.SemaphoreType.DMA] * 2,
    compiler_params=pltpu.CompilerParams(use_tc_tiling_on_sc=False),
)


def _sc_deg_body(dstx, deg, dst_v, zd_v, ones_v, deg_sh):
    cid = lax.axis_index("c")
    sid = lax.axis_index("s")
    wid = cid * NS + sid

    pltpu.sync_copy(dstx.at[wid], dst_v)

    z16 = jnp.zeros((16,), jnp.float32)
    o16 = jnp.ones((16,), jnp.float32)

    def zdrow(i, _):
        zd_v[i, pl.ds(0, 16)] = z16
        return 0
    lax.fori_loop(0, RPT, zdrow, 0)

    def orow(i, _):
        ones_v[i, pl.ds(0, 16)] = o16
        return 0
    lax.fori_loop(0, C, orow, 0)

    pltpu.sync_copy(zd_v, deg_sh.at[pl.ds(sid * RPT, RPT)])
    plsc.subcore_barrier()

    def step(j, _):
        pltpu.sync_copy(ones_v, deg_sh.at[dst_v.at[j]], add=True)
        return 0
    lax.fori_loop(0, CH, step, 0)
    plsc.subcore_barrier()

    sl = pl.ds(sid * RPT, RPT)
    pltpu.sync_copy(deg_sh.at[sl], deg.at[cid, sl])


_sc_deg = pl.kernel(
    _sc_deg_body,
    out_type=[jax.ShapeDtypeStruct((NC, NPAD, 16), jnp.float32)],
    mesh=plsc.VectorSubcoreMesh(core_axis_name="c", subcore_axis_name="s"),
    scratch_types=[
        pltpu.VMEM((CH, C), jnp.int32),
        pltpu.VMEM((RPT, 16), jnp.float32),
        pltpu.VMEM((C, 16), jnp.float32),
        pltpu.VMEM_SHARED((NPAD, 16), jnp.float32),
    ],
    compiler_params=pltpu.CompilerParams(use_tc_tiling_on_sc=False),
)


def _mm_body(x_ref, w_ref, ya, yb, r):
    yw = jnp.dot(x_ref[...], w_ref[...], preferred_element_type=jnp.float32)
    ya[...] = yw[:, :H]
    yb[...] = yw[:, H:D]
    r[...] = yw[:, D:]


def _matmul(x, w):
    return pl.pallas_call(
        _mm_body,
        grid=(N // BN,),
        in_specs=[pl.BlockSpec((BN, D), lambda i: (i, 0)),
                  pl.BlockSpec((D, 2 * D), lambda i: (0, 0))],
        out_specs=[pl.BlockSpec((BN, H), lambda i: (i, 0)),
                   pl.BlockSpec((BN, H), lambda i: (i, 0)),
                   pl.BlockSpec((BN, D), lambda i: (i, 0))],
        out_shape=[jax.ShapeDtypeStruct((N, H), jnp.float32),
                   jax.ShapeDtypeStruct((N, H), jnp.float32),
                   jax.ShapeDtypeStruct((N, D), jnp.float32)],
    )(x, w)


def _comb_mm_body(p0, p1, d0, d1, r1, b, w, ya, yb, r2):
    deg = jnp.maximum(d0[...] + d1[...], 1.0)[:, 0:1]
    hh = jnp.maximum((p0[...] + p1[...]) / deg + b[...] + r1[...], 0.0)
    yw = jnp.dot(hh, w[...], preferred_element_type=jnp.float32)
    ya[...] = yw[:, :H]
    yb[...] = yw[:, H:D]
    r2[...] = yw[:, D:]


def _combine_mm(p0, p1, d0, d1, r1, b, w):
    return pl.pallas_call(
        _comb_mm_body,
        grid=(N // BN,),
        in_specs=[pl.BlockSpec((BN, D), lambda i: (i, 0)),
                  pl.BlockSpec((BN, D), lambda i: (i, 0)),
                  pl.BlockSpec((BN, 16), lambda i: (i, 0)),
                  pl.BlockSpec((BN, 16), lambda i: (i, 0)),
                  pl.BlockSpec((BN, D), lambda i: (i, 0)),
                  pl.BlockSpec((1, D), lambda i: (0, 0)),
                  pl.BlockSpec((D, 2 * D), lambda i: (0, 0))],
        out_specs=[pl.BlockSpec((BN, H), lambda i: (i, 0)),
                   pl.BlockSpec((BN, H), lambda i: (i, 0)),
                   pl.BlockSpec((BN, D), lambda i: (i, 0))],
        out_shape=[jax.ShapeDtypeStruct((N, H), jnp.float32),
                   jax.ShapeDtypeStruct((N, H), jnp.float32),
                   jax.ShapeDtypeStruct((N, D), jnp.float32)],
    )(p0, p1, d0, d1, r1, b, w)


def _comb_pool_body(p0, p1, d0, d1, r2, b, wo, bo, out, acc):
    i = pl.program_id(0)
    deg = jnp.maximum(d0[...] + d1[...], 1.0)[:, 0:1]
    hh = jnp.maximum((p0[...] + p1[...]) / deg + b[...] + r2[...], 0.0)
    s = jnp.sum(hh, axis=0, keepdims=True)

    @pl.when(i == 0)
    def _():
        acc[...] = jnp.zeros_like(acc)

    acc[0:1, :] += s
    out[...] = (jnp.dot(acc[0:1, :] * (1.0 / N), wo[...],
                        preferred_element_type=jnp.float32) + bo[...])


def _combine_pool(p0, p1, d0, d1, r2, b, wo, bo):
    return pl.pallas_call(
        _comb_pool_body,
        grid=(N // BN,),
        in_specs=[pl.BlockSpec((BN, D), lambda i: (i, 0)),
                  pl.BlockSpec((BN, D), lambda i: (i, 0)),
                  pl.BlockSpec((BN, 16), lambda i: (i, 0)),
                  pl.BlockSpec((BN, 16), lambda i: (i, 0)),
                  pl.BlockSpec((BN, D), lambda i: (i, 0)),
                  pl.BlockSpec((1, D), lambda i: (0, 0)),
                  pl.BlockSpec((D, O), lambda i: (0, 0)),
                  pl.BlockSpec((1, O), lambda i: (0, 0))],
        out_specs=pl.BlockSpec((1, O), lambda i: (0, 0)),
        out_shape=jax.ShapeDtypeStruct((1, O), jnp.float32),
        scratch_shapes=[pltpu.VMEM((8, D), jnp.float32)],
    )(p0, p1, d0, d1, r2, b, wo, bo)


def kernel(x, edge_index, W1l, b1, W1r, W2l, b2, W2r, Wo, bo):
    src = edge_index[0].astype(jnp.int32)
    dst = edge_index[1].astype(jnp.int32)
    pad = EPAD - E
    srcx = jnp.concatenate([src, jnp.arange(pad, dtype=jnp.int32)]).reshape(NW, CH, C)
    dstx = jnp.concatenate([dst, jnp.full((pad,), N, jnp.int32)]).reshape(NW, CH, C)

    y1a, y1b, r1 = _matmul(x, jnp.concatenate([W1l, W1r], axis=1))
    (degs,) = _sc_deg(dstx)
    (parts1,) = _sc_segsum(y1a, y1b, srcx, dstx)
    y2a, y2b, r2 = _combine_mm(parts1[0], parts1[1], degs[0], degs[1],
                               r1, b1.reshape(1, D),
                               jnp.concatenate([W2l, W2r], axis=1))
    (parts2,) = _sc_segsum(y2a, y2b, srcx, dstx)
    return _combine_pool(parts2[0], parts2[1], degs[0], degs[1], r2,
                         b2.reshape(1, D), Wo, bo.reshape(1, O))
